# baseline (speedup 1.0000x reference)
import jax
import jax.numpy as jnp
from jax.experimental import pallas as pl
from jax.experimental.pallas import tpu as pltpu

_DILATION = 2


def _plane_gather(x_hbm, o_hbm, sems):
    nk = o_hbm.shape[0]
    for j in range(nk):
        pltpu.make_async_copy(
            x_hbm.at[_DILATION * j], o_hbm.at[j], sems.at[j]).start()
    for j in range(nk):
        pltpu.make_async_copy(
            x_hbm.at[_DILATION * j], o_hbm.at[j], sems.at[j]).wait()


def kernel(edge_index):
    two, n, kd = edge_index.shape
    k = kd // _DILATION
    xt = jnp.transpose(edge_index, (2, 0, 1))
    out_t = pl.pallas_call(
        _plane_gather,
        in_specs=[pl.BlockSpec(memory_space=pltpu.MemorySpace.HBM)],
        out_specs=pl.BlockSpec(memory_space=pltpu.MemorySpace.HBM),
        out_shape=jax.ShapeDtypeStruct((k, two, n), edge_index.dtype),
        scratch_shapes=[pltpu.SemaphoreType.DMA((k,))],
    )(xt)
    return jnp.transpose(out_t, (1, 2, 0))


# speedup vs baseline: 18.6350x; 18.6350x over previous
import jax
import jax.numpy as jnp
from jax.experimental import pallas as pl
from jax.experimental.pallas import tpu as pltpu

_DILATION = 2


def _plane_gather(x_ref, o_hbm, sem):
    j = pl.program_id(0)
    cp = pltpu.make_async_copy(x_ref, o_hbm.at[pl.ds(j, 1)], sem)
    cp.start()
    cp.wait()


def kernel(edge_index):
    two, n, kd = edge_index.shape
    k = kd // _DILATION
    xt = jnp.transpose(edge_index, (2, 0, 1))
    out_t = pl.pallas_call(
        _plane_gather,
        grid=(k,),
        in_specs=[pl.BlockSpec((1, two, n), lambda j: (_DILATION * j, 0, 0))],
        out_specs=pl.BlockSpec(memory_space=pl.MemorySpace.ANY),
        out_shape=jax.ShapeDtypeStruct((k, two, n), edge_index.dtype),
        scratch_shapes=[pltpu.SemaphoreType.DMA],
    )(xt)
    return jnp.transpose(out_t, (1, 2, 0))
